# baseline (speedup 1.0000x reference)
import functools

import jax
import jax.numpy as jnp
from jax.experimental import pallas as pl
from jax.experimental.pallas import tpu as pltpu


def _router_kernel(x_ref, w1_ref, b1_ref, w2_ref, b2_ref, out_ref,
                   h0, h1, logits_acc, *, m_steps, k_steps, bk):
    m = pl.program_id(0)
    k = pl.program_id(1)
    parity = jax.lax.rem(m, 2)

    @pl.when(m < m_steps)
    def _matmul():
        @pl.when(parity == 0)
        def _even():
            @pl.when(k == 0)
            def _init():
                h0[...] = jnp.zeros_like(h0)
            h0[...] += jnp.dot(x_ref[...], w1_ref[...],
                               preferred_element_type=jnp.float32)

        @pl.when(parity == 1)
        def _odd():
            @pl.when(k == 0)
            def _init():
                h1[...] = jnp.zeros_like(h1)
            h1[...] += jnp.dot(x_ref[...], w1_ref[...],
                               preferred_element_type=jnp.float32)

    @pl.when(m > 0)
    def _epilogue():
        def _slice_epi(h_ref):
            h = h_ref[:, pl.ds(k * bk, bk)] + b1_ref[...]
            hg = 0.5 * h * (1.0 + jax.lax.erf(h * 0.7071067811865476))
            part = jnp.dot(hg, w2_ref[...], preferred_element_type=jnp.float32)

            @pl.when(k == 0)
            def _init_logits():
                logits_acc[...] = part + b2_ref[...]

            @pl.when(k > 0)
            def _acc_logits():
                logits_acc[...] += part

            @pl.when(k == k_steps - 1)
            def _softmax():
                logits = logits_acc[...]
                mx = jnp.max(logits, axis=1, keepdims=True)
                e = jnp.exp(logits - mx)
                out_ref[...] = e / jnp.sum(e, axis=1, keepdims=True)

        @pl.when(parity == 1)
        def _from_even():
            _slice_epi(h0)

        @pl.when(parity == 0)
        def _from_odd():
            _slice_epi(h1)


@jax.jit
def kernel(x, W1, b1, W2, b2):
    M, K = x.shape
    _, N = W1.shape
    E = W2.shape[1]

    bm = min(1024, M)
    bk = min(512, K)
    m_steps = M // bm
    k_steps = K // bk
    grid = (m_steps + 1, k_steps)

    b1r = b1.reshape(1, N)
    b2r = b2.reshape(1, E)
    last_m = m_steps - 1

    return pl.pallas_call(
        functools.partial(_router_kernel, m_steps=m_steps, k_steps=k_steps,
                          bk=bk),
        grid=grid,
        in_specs=[
            pl.BlockSpec((bm, bk), lambda m, k: (jnp.minimum(m, last_m), k)),
            pl.BlockSpec((bk, N), lambda m, k: (k, 0)),
            pl.BlockSpec((1, bk), lambda m, k: (0, k)),
            pl.BlockSpec((bk, E), lambda m, k: (k, 0)),
            pl.BlockSpec((1, E), lambda m, k: (0, 0)),
        ],
        out_specs=pl.BlockSpec((bm, E), lambda m, k: (jnp.maximum(m - 1, 0), 0)),
        out_shape=jax.ShapeDtypeStruct((M, E), jnp.float32),
        scratch_shapes=[
            pltpu.VMEM((bm, N), jnp.float32),
            pltpu.VMEM((bm, N), jnp.float32),
            pltpu.VMEM((bm, E), jnp.float32),
        ],
        compiler_params=pltpu.CompilerParams(
            dimension_semantics=("arbitrary", "arbitrary")),
    )(x, W1, b1r, W2, b2r)
